# Initial kernel scaffold; baseline (speedup 1.0000x reference)
import jax
import jax.numpy as jnp
from jax.experimental import pallas as pl


def kernel(nv, ns, x, edge_index, ms1_w, ms1_b, ms2_w, ms2_b, mv_w, mv_b, us1_w, us1_b, us2_w, us2_b):
    raise NotImplementedError("write your pallas kernel here")


# speedup vs baseline: 5.5322x; 5.5322x over previous
import functools

import jax
import jax.numpy as jnp
from jax import lax
from jax.experimental import pallas as pl
from jax.experimental.pallas import tpu as pltpu
from jax.experimental.pallas import tpu_sc as plsc

_EPS = 1e-05
_RC = 5.0
_L = 6
_LOG2 = 0.6931471805599453

_N = 10000
_E = 160000
_D = 128
_NC = 2
_NS = 16
_NW = _NC * _NS
_EPAD = 163840
_RPT32 = _EPAD // _NW
_RPT16 = _EPAD // _NS
_CH = 128
_NACC = 10112


def _mlp_body(s_ref, w1_ref, b1_ref, w2_ref, b2_ref, out_ref):
    h = jnp.dot(s_ref[...], w1_ref[...], preferred_element_type=jnp.float32)
    h = jax.nn.softplus(h + b1_ref[...]) - _LOG2
    out_ref[...] = (
        jnp.dot(h, w2_ref[...], preferred_element_type=jnp.float32) + b2_ref[...]
    )


def _node_mlp(s, w1, b1, w2, b2, blk=1000):
    n, d = s.shape
    d3 = w2.shape[1]
    return pl.pallas_call(
        _mlp_body,
        grid=(n // blk,),
        in_specs=[
            pl.BlockSpec((blk, d), lambda i: (i, 0)),
            pl.BlockSpec((d, d), lambda i: (0, 0)),
            pl.BlockSpec((1, d), lambda i: (0, 0)),
            pl.BlockSpec((d, d3), lambda i: (0, 0)),
            pl.BlockSpec((1, d3), lambda i: (0, 0)),
        ],
        out_specs=pl.BlockSpec((blk, d3), lambda i: (i, 0)),
        out_shape=jax.ShapeDtypeStruct((n, d3), jnp.float32),
    )(s, w1, b1.reshape(1, d), w2, b2.reshape(1, d3))


def _edge_body(v0_ref, v1_ref, v2_ref, g_ref, mw_ref, mb_ref, out_ref):
    v0 = v0_ref[...]
    v1 = v1_ref[...]
    v2 = v2_ref[...]
    r = jnp.sqrt(v0 * v0 + v1 * v1 + v2 * v2 + _EPS)
    wl = mb_ref[...]
    scale = jnp.sqrt(2.0 / _RC)
    for l in range(_L):
        basis_l = scale * jnp.sin((l + 1) * jnp.pi * r / _RC) / r
        wl = wl + basis_l * mw_ref[l : l + 1, :]
    w = 0.5 * (jnp.cos(jnp.pi * wl / _RC) + 1.0) * (wl < _RC).astype(jnp.float32)
    wv = w[:, :_D]
    ws = w[:, _D : 2 * _D]
    wr = w[:, 2 * _D :]
    pv = g_ref[3] * wv
    ps = g_ref[4] * ws
    pr = g_ref[5] * wr
    out_ref[0] = g_ref[0] * pv + pr * (v0 / r)
    out_ref[1] = g_ref[1] * pv + pr * (v1 / r)
    out_ref[2] = g_ref[2] * pv + pr * (v2 / r)
    out_ref[3] = ps


def _edge_payload(vcols, g6, mv_w8, mv_b, blk=512):
    grid = (_EPAD // blk,)
    return pl.pallas_call(
        _edge_body,
        grid=grid,
        in_specs=[
            pl.BlockSpec((blk, 1), lambda i: (i, 0)),
            pl.BlockSpec((blk, 1), lambda i: (i, 0)),
            pl.BlockSpec((blk, 1), lambda i: (i, 0)),
            pl.BlockSpec((6, blk, _D), lambda i: (0, i, 0)),
            pl.BlockSpec((8, 3 * _D), lambda i: (0, 0)),
            pl.BlockSpec((1, 3 * _D), lambda i: (0, 0)),
        ],
        out_specs=pl.BlockSpec((4, blk, _D), lambda i: (0, i, 0)),
        out_shape=jax.ShapeDtypeStruct((4, _EPAD, _D), jnp.float32),
    )(vcols[0], vcols[1], vcols[2], g6, mv_w8, mv_b.reshape(1, 3 * _D))


def _update_body(nv0_ref, nv1_ref, nv2_ref, dv_ref, ns_ref, ds_ref,
                 w1_ref, b1_ref, w2_ref, b2_ref,
                 v0_ref, v1_ref, v2_ref, sn_ref, s2_ref):
    v0_ref[...] = nv0_ref[...] + dv_ref[0]
    v1_ref[...] = nv1_ref[...] + dv_ref[1]
    v2_ref[...] = nv2_ref[...] + dv_ref[2]
    s_new = ns_ref[...] + ds_ref[...]
    sn_ref[...] = s_new
    h = jnp.dot(s_new, w1_ref[...], preferred_element_type=jnp.float32)
    h = jax.nn.softplus(h + b1_ref[...]) - _LOG2
    s2_ref[...] = (
        jnp.dot(h, w2_ref[...], preferred_element_type=jnp.float32) + b2_ref[...]
    )


def _node_update(nv0, nv1, nv2, dv3, ns, ds_, w1, b1, w2, b2, blk=1000):
    d = _D
    d3 = 3 * _D
    nblk = pl.BlockSpec((blk, d), lambda i: (i, 0))
    return pl.pallas_call(
        _update_body,
        grid=(_N // blk,),
        in_specs=[
            nblk, nblk, nblk,
            pl.BlockSpec((3, blk, d), lambda i: (0, i, 0)),
            nblk, nblk,
            pl.BlockSpec((d, d), lambda i: (0, 0)),
            pl.BlockSpec((1, d), lambda i: (0, 0)),
            pl.BlockSpec((d, d3), lambda i: (0, 0)),
            pl.BlockSpec((1, d3), lambda i: (0, 0)),
        ],
        out_specs=[nblk, nblk, nblk, nblk,
                   pl.BlockSpec((blk, d3), lambda i: (i, 0))],
        out_shape=[
            jax.ShapeDtypeStruct((_N, d), jnp.float32),
            jax.ShapeDtypeStruct((_N, d), jnp.float32),
            jax.ShapeDtypeStruct((_N, d), jnp.float32),
            jax.ShapeDtypeStruct((_N, d), jnp.float32),
            jax.ShapeDtypeStruct((_N, d3), jnp.float32),
        ],
    )(nv0, nv1, nv2, dv3, ns, ds_, w1, b1.reshape(1, d), w2,
      b2.reshape(1, d3))


def _final_body(v0_ref, v1_ref, v2_ref, sn_ref, u_ref,
                ov0_ref, ov1_ref, ov2_ref, os_ref):
    dn = jnp.maximum(u_ref[6], 1.0)
    uv0 = u_ref[0] / dn
    uv1 = u_ref[1] / dn
    uv2 = u_ref[2] / dn
    avv = u_ref[3] / dn
    asv = u_ref[4] / dn
    ass = u_ref[5] / dn
    ov0_ref[...] = v0_ref[...] + uv0 * avv
    ov1_ref[...] = v1_ref[...] + uv1 * avv
    ov2_ref[...] = v2_ref[...] + uv2 * avv
    su = uv0 * uv0 + uv1 * uv1 + uv2 * uv2
    os_ref[...] = sn_ref[...] + (su / (su + _EPS)) * asv + ass


def _final_combine(v0, v1, v2, s_new, u7, blk=400):
    d = _D
    nblk = pl.BlockSpec((blk, d), lambda i: (i, 0))
    return pl.pallas_call(
        _final_body,
        grid=(_N // blk,),
        in_specs=[
            nblk, nblk, nblk, nblk,
            pl.BlockSpec((7, blk, d), lambda i: (0, i, 0)),
        ],
        out_specs=[nblk, nblk, nblk, nblk],
        out_shape=[jax.ShapeDtypeStruct((_N, d), jnp.float32)] * 4,
    )(v0, v1, v2, s_new, u7)


@functools.lru_cache(maxsize=1)
def _sc_mesh():
    return plsc.VectorSubcoreMesh(core_axis_name="c", subcore_axis_name="s")


def _g6_body(tab_ref, idx_ref, out_ref, idx_v, rows_v, sem):
    wid = lax.axis_index("c") * _NS + lax.axis_index("s")

    def it(t, carry):
        sl = t // 40
        i = t - sl * 40
        off = sl * _EPAD + wid * _RPT32 + i * _CH
        pltpu.sync_copy(idx_ref.at[pl.ds(off, _CH)], idx_v)
        pltpu.async_copy(tab_ref.at[idx_v], rows_v, sem).wait()
        pltpu.sync_copy(rows_v, out_ref.at[pl.ds(off, _CH)])
        return carry

    lax.fori_loop(0, 6 * 40, it, 0)


def _sc_gather6(tab, idx):
    return pl.kernel(
        _g6_body,
        out_type=jax.ShapeDtypeStruct((6 * _EPAD, _D), jnp.float32),
        mesh=_sc_mesh(),
        scratch_types=[
            pltpu.VMEM((_CH,), jnp.int32),
            pltpu.VMEM((_CH, _D), jnp.float32),
            pltpu.SemaphoreType.DMA,
        ],
    )(tab, idx)


def _zero_acc(zeros_ref, acc, s):
    def zit(q, carry):
        cz = s + _NS * q

        @pl.when(cz < _NACC // _CH)
        def _():
            pltpu.sync_copy(zeros_ref, acc.at[pl.ds(cz * _CH, _CH)])

        return carry

    lax.fori_loop(0, 5, zit, 0)


def _dump_acc(acc, out_ref, sl, s, rows_v):
    def dit(q, carry):
        cz = s + _NS * q

        @pl.when(cz < _NACC // _CH)
        def _():
            pltpu.sync_copy(acc.at[pl.ds(cz * _CH, _CH)], rows_v)
            pltpu.sync_copy(rows_v, out_ref.at[pl.ds(sl * _NACC + cz * _CH, _CH)])

        return carry

    lax.fori_loop(0, 5, dit, 0)


def _scat4_body(pay_ref, dst_ref, zeros_ref, out_ref, idx_v, rows_v, sem, acc):
    c = lax.axis_index("c")
    s = lax.axis_index("s")
    for j in range(2):
        sl = 2 * c + j
        _zero_acc(zeros_ref, acc, s)
        plsc.subcore_barrier()

        def ait(t, carry):
            eoff = s * _RPT16 + t * _CH
            pltpu.sync_copy(dst_ref.at[pl.ds(eoff, _CH)], idx_v)
            pltpu.sync_copy(pay_ref.at[pl.ds(sl * _EPAD + eoff, _CH)], rows_v)
            pltpu.sync_copy(rows_v, acc.at[idx_v], add=True)
            return carry

        lax.fori_loop(0, _RPT16 // _CH, ait, 0)
        plsc.subcore_barrier()
        _dump_acc(acc, out_ref, sl, s, rows_v)
        plsc.subcore_barrier()


def _sc_scatter4(pay, dst_s, zeros128):
    return pl.kernel(
        _scat4_body,
        out_type=jax.ShapeDtypeStruct((4 * _NACC, _D), jnp.float32),
        mesh=_sc_mesh(),
        scratch_types=[
            pltpu.VMEM((_CH,), jnp.int32),
            pltpu.VMEM((_CH, _D), jnp.float32),
            pltpu.SemaphoreType.DMA,
            pltpu.VMEM_SHARED((_NACC, _D), jnp.float32),
        ],
    )(pay, dst_s, zeros128)


def _sgs7_body(tab_ref, idx_ref, dst_ref, zeros_ref, out_ref,
               idx_v, didx_v, rows_v, sem, acc):
    c = lax.axis_index("c")
    s = lax.axis_index("s")
    for j in range(4):
        sl = jnp.minimum(4 * c + j, 6)
        _zero_acc(zeros_ref, acc, s)
        plsc.subcore_barrier()

        def ait(t, carry):
            eoff = s * _RPT16 + t * _CH
            pltpu.sync_copy(idx_ref.at[pl.ds(sl * _EPAD + eoff, _CH)], idx_v)
            pltpu.async_copy(tab_ref.at[idx_v], rows_v, sem).wait()
            pltpu.sync_copy(dst_ref.at[pl.ds(eoff, _CH)], didx_v)
            pltpu.sync_copy(rows_v, acc.at[didx_v], add=True)
            return carry

        lax.fori_loop(0, _RPT16 // _CH, ait, 0)
        plsc.subcore_barrier()

        @pl.when(4 * c + j < 7)
        def _():
            _dump_acc(acc, out_ref, sl, s, rows_v)

        plsc.subcore_barrier()


def _sc_sgs7(tab, idx, dst_s, zeros128):
    return pl.kernel(
        _sgs7_body,
        out_type=jax.ShapeDtypeStruct((7 * _NACC, _D), jnp.float32),
        mesh=_sc_mesh(),
        scratch_types=[
            pltpu.VMEM((_CH,), jnp.int32),
            pltpu.VMEM((_CH,), jnp.int32),
            pltpu.VMEM((_CH, _D), jnp.float32),
            pltpu.SemaphoreType.DMA,
            pltpu.VMEM_SHARED((_NACC, _D), jnp.float32),
        ],
    )(tab, idx, dst_s, zeros128)


def kernel(nv, ns, x, edge_index, ms1_w, ms1_b, ms2_w, ms2_b, mv_w, mv_b,
           us1_w, us1_b, us2_w, us2_b):
    src = edge_index[0]
    dst = edge_index[1]
    pad = _EPAD - _E
    src_p = jnp.concatenate([src, jnp.zeros((pad,), jnp.int32)])
    dst_s = jnp.concatenate([dst, jnp.full((pad,), _N, jnp.int32)])

    vec = x[src] - x[dst]
    vcol = jnp.pad(vec, ((0, pad), (0, 0)))
    vcols = [vcol[:, k : k + 1] for k in range(3)]

    phi_n = _node_mlp(ns, ms1_w, ms1_b, ms2_w, ms2_b)
    tab1 = jnp.concatenate(
        [nv[:, :, 0], nv[:, :, 1], nv[:, :, 2],
         phi_n[:, :_D], phi_n[:, _D : 2 * _D], phi_n[:, 2 * _D :]], axis=0)
    offs = (jnp.arange(7, dtype=jnp.int32) * _N)[:, None]
    idx7 = (src_p[None, :] + offs).reshape(-1)
    g6 = _sc_gather6(tab1, idx7[: 6 * _EPAD]).reshape(6, _EPAD, _D)

    mv_w8 = jnp.pad(mv_w, ((0, 2), (0, 0)))
    pay = _edge_payload(vcols, g6, mv_w8, mv_b)

    zeros128 = jnp.zeros((_CH, _D), jnp.float32)
    scat = _sc_scatter4(pay.reshape(4 * _EPAD, _D), dst_s, zeros128)
    scat = scat.reshape(4, _NACC, _D)[:, :_N]

    v0, v1, v2, s_new, s2 = _node_update(
        nv[:, :, 0], nv[:, :, 1], nv[:, :, 2], scat[:3], ns, scat[3],
        us1_w, us1_b, us2_w, us2_b)

    tab2 = jnp.concatenate(
        [v0, v1, v2, s2[:, :_D], s2[:, _D : 2 * _D], s2[:, 2 * _D :],
         jnp.ones((_N, _D), jnp.float32)], axis=0)
    u7 = _sc_sgs7(tab2, idx7, dst_s, zeros128)
    u7 = u7.reshape(7, _NACC, _D)[:, :_N]

    ov0, ov1, ov2, os_ = _final_combine(v0, v1, v2, s_new, u7)
    v_out = jnp.stack([ov0, ov1, ov2], axis=-1)
    return (v_out, os_)
